# baseline (device time: 46734 ns/iter reference)
import os

import jax
import jax.numpy as jnp
from jax import lax
from jax.experimental import pallas as pl
from jax.experimental.pallas import tpu as pltpu

_ABL = os.environ.get("KABL", "")

N_SLABS = 8
NS = 4
CHUNK = 64
RS_OFF = (0, 8, 12, 14)
N_STEPS = 4


def kernel(dy, W):
    m, k_shard = dy.shape
    d = W.shape[0]
    slab = k_shard // N_SLABS
    csz = d // NS

    def body(dy_hbm, w_hbm, out_ref, a_buf, b_buf, a16, b16, acc, rbuf,
             in_sems, send_sems, recv_sems):
        x = lax.axis_index("x")
        y = lax.axis_index("y")
        z = lax.axis_index("z")
        c = x * 4 + z

        cpa = pltpu.make_async_copy(
            dy_hbm.at[:, pl.ds(c * slab, slab)], a_buf, in_sems.at[0])
        cpb = pltpu.make_async_copy(
            w_hbm.at[:, pl.ds(c * slab, slab)], b_buf, in_sems.at[1])
        cpa.start()
        cpb.start()

        neighbors = [
            (1 - x, y, z),
            (x, 1 - y, z),
            (x, y, jnp.bitwise_xor(z, 1)),
            (x, y, jnp.bitwise_xor(z, 2)),
        ]
        bsem = pltpu.get_barrier_semaphore()
        for pid in neighbors:
            pl.semaphore_signal(bsem, inc=1, device_id=pid,
                                device_id_type=pl.DeviceIdType.MESH)
        pl.semaphore_wait(bsem, 4)

        info = {
            8: (neighbors[0], x),
            4: (neighbors[1], y),
            1: (neighbors[2], z & 1),
            2: (neighbors[3], z >> 1),
        }
        orders = [
            (8, 4, 1, 2),
            (8, 1, 4, 2),
            (4, 8, 2, 1),
            (4, 2, 8, 1),
        ][:NS]
        col = [pl.ds(j * csz, csz) for j in range(NS)]
        drain = []

        def start_rs(s, j, lo_j):
            half = 8 >> s
            rows = CHUNK if _ABL == "tiny" else half * CHUNK
            pid, mybit = info[orders[j][s]]
            keep_lo = lo_j + mybit * half
            send_lo = lo_j + (1 - mybit) * half
            rdma = pltpu.make_async_remote_copy(
                src_ref=acc.at[pl.ds(send_lo * CHUNK, rows), col[j]],
                dst_ref=rbuf.at[pl.ds(RS_OFF[s] * CHUNK, rows), col[j]],
                send_sem=send_sems.at[s, j],
                recv_sem=recv_sems.at[s, j],
                device_id=pid,
                device_id_type=pl.DeviceIdType.MESH,
            )
            rdma.start()
            return rdma, keep_lo

        def start_ag(s, j, lo_j):
            rows = CHUNK if _ABL == "tiny" else (1 << s) * CHUNK
            pid, mybit = info[orders[j][N_STEPS - 1 - s]]
            rdma = pltpu.make_async_remote_copy(
                src_ref=acc.at[pl.ds(lo_j * CHUNK, rows), col[j]],
                dst_ref=acc.at[pl.ds(lo_j * CHUNK, rows), col[j]],
                send_sem=send_sems.at[N_STEPS + s, j],
                recv_sem=recv_sems.at[N_STEPS + s, j],
                device_id=pid,
                device_id_type=pl.DeviceIdType.MESH,
            )
            rdma.start()
            return rdma, mybit

        cpa.wait()
        cpb.wait()
        a16[...] = a_buf[...].astype(jnp.bfloat16)
        b16[...] = b_buf[...].astype(jnp.bfloat16)

        lo = [jnp.int32(0)] * NS
        R, keep = [None] * NS, [None] * NS
        abit = [None] * NS
        for j in range(NS):
            acc[:, col[j]] = lax.dot_general(
                a16[...], b16[pl.ds(j * csz, csz), :],
                dimension_numbers=(((1,), (1,)), ((), ())),
                preferred_element_type=jnp.float32,
            ).astype(jnp.bfloat16)
            R[j], keep[j] = start_rs(0, j, lo[j])

        for s in range(N_STEPS):
            rows = CHUNK if _ABL == "tiny" else (8 >> s) * CHUNK
            off = RS_OFF[s] * CHUNK
            for j in range(NS):
                R[j].wait_recv()
                drain.append(R[j])
                acc[pl.ds(keep[j] * CHUNK, rows), col[j]] += (
                    rbuf[pl.ds(off, rows), col[j]])
                lo[j] = keep[j]
                if s < N_STEPS - 1:
                    R[j], keep[j] = start_rs(s + 1, j, lo[j])
                elif _ABL != "rsonly":
                    R[j], abit[j] = start_ag(0, j, lo[j])
                    out_ref[pl.ds(lo[j] * CHUNK, CHUNK), col[j]] = (
                        acc[pl.ds(lo[j] * CHUNK, CHUNK), col[j]]
                        .astype(jnp.float32))

        if _ABL == "rsonly":
            out_ref[...] = acc[...].astype(jnp.float32)
            for rdma in drain:
                rdma.wait_send()
            return

        for s in range(N_STEPS):
            rows = (1 << s) * CHUNK
            for j in range(NS):
                R[j].wait_recv()
                drain.append(R[j])
                recv_lo = lo[j] + (1 - 2 * abit[j]) * (1 << s)
                lo[j] = lo[j] - abit[j] * (1 << s)
                if s < N_STEPS - 1:
                    R[j], abit[j] = start_ag(s + 1, j, lo[j])
                out_ref[pl.ds(recv_lo * CHUNK, rows), col[j]] = (
                    acc[pl.ds(recv_lo * CHUNK, rows), col[j]]
                    .astype(jnp.float32))

        for rdma in drain:
            rdma.wait_send()

    return pl.pallas_call(
        body,
        out_shape=jax.ShapeDtypeStruct((m, d), jnp.float32),
        in_specs=[
            pl.BlockSpec(memory_space=pltpu.MemorySpace.HBM),
            pl.BlockSpec(memory_space=pltpu.MemorySpace.HBM),
        ],
        out_specs=pl.BlockSpec(memory_space=pltpu.VMEM),
        scratch_shapes=[
            pltpu.VMEM((m, slab), jnp.float32),
            pltpu.VMEM((m, slab), jnp.float32),
            pltpu.VMEM((m, slab), jnp.bfloat16),
            pltpu.VMEM((m, slab), jnp.bfloat16),
            pltpu.VMEM((m, d), jnp.bfloat16),
            pltpu.VMEM((15 * CHUNK, d), jnp.bfloat16),
            pltpu.SemaphoreType.DMA((2,)),
            pltpu.SemaphoreType.DMA((2 * N_STEPS, NS)),
            pltpu.SemaphoreType.DMA((2 * N_STEPS, NS)),
        ],
        compiler_params=pltpu.CompilerParams(
            collective_id=0,
            vmem_limit_bytes=100 * 1024 * 1024,
        ),
    )(dy, W)


# device time: 44442 ns/iter; 1.0516x vs baseline; 1.0516x over previous
import os

import jax
import jax.numpy as jnp
from jax import lax
from jax.experimental import pallas as pl
from jax.experimental.pallas import tpu as pltpu

N_SLABS = 8
NS = int(os.environ.get("KNS", "4"))
CHUNK = 64
N_LVL = 6
RS_OFF = (0, 8, 12, 16)
RS_ROWS = (8, 4, 4, 4)


def kernel(dy, W):
    m, k_shard = dy.shape
    d = W.shape[0]
    slab = k_shard // N_SLABS
    csz = d // NS

    def body(dy_hbm, w_hbm, out_ref, a_buf, b_buf, a16, b16, acc, rbuf,
             in_sems, send_sems, recv_sems):
        x = lax.axis_index("x")
        y = lax.axis_index("y")
        z = lax.axis_index("z")
        c = x * 4 + z

        cpa = pltpu.make_async_copy(
            dy_hbm.at[:, pl.ds(c * slab, slab)], a_buf, in_sems.at[0])
        cpb = pltpu.make_async_copy(
            w_hbm.at[:, pl.ds(c * slab, slab)], b_buf, in_sems.at[1])
        cpa.start()
        cpb.start()

        neighbors = [
            (1 - x, y, z),
            (x, 1 - y, z),
            (x, y, jnp.bitwise_xor(z, 1)),
            (x, y, jnp.bitwise_xor(z, 2)),
        ]
        bsem = pltpu.get_barrier_semaphore()
        for pid in neighbors:
            pl.semaphore_signal(bsem, inc=1, device_id=pid,
                                device_id_type=pl.DeviceIdType.MESH)
        pl.semaphore_wait(bsem, 4)

        info = {
            8: (neighbors[0], x),
            4: (neighbors[1], y),
            1: (neighbors[2], z & 1),
            2: (neighbors[3], z >> 1),
        }
        orders = [
            (8, 4, 1, 2),
            (4, 8, 2, 1),
            (8, 4, 2, 1),
            (4, 8, 1, 2),
        ][:NS]
        col = [pl.ds(j * csz, csz) for j in range(NS)]
        lvl_mask = [
            [o[0], o[1], o[2], o[3], o[1], o[0]] for o in orders
        ]
        drain = []

        def start_lvl(lv, j, lo_j):
            pid, mybit = info[lvl_mask[j][lv]]
            if lv < 2:
                half = 8 >> lv
                send_lo = lo_j + (1 - mybit) * half
                keep_lo = lo_j + mybit * half
                src = acc.at[pl.ds(send_lo * CHUNK, half * CHUNK), col[j]]
                dst = rbuf.at[pl.ds(RS_OFF[lv] * CHUNK, half * CHUNK), col[j]]
                nxt = keep_lo
            elif lv < 4:
                src = acc.at[pl.ds(lo_j * CHUNK, 4 * CHUNK), col[j]]
                dst = rbuf.at[pl.ds(RS_OFF[lv] * CHUNK, 4 * CHUNK), col[j]]
                nxt = lo_j
            else:
                rows = 4 * CHUNK if lv == 4 else 8 * CHUNK
                src = acc.at[pl.ds(lo_j * CHUNK, rows), col[j]]
                dst = acc.at[pl.ds(lo_j * CHUNK, rows), col[j]]
                nxt = mybit
            rdma = pltpu.make_async_remote_copy(
                src_ref=src, dst_ref=dst,
                send_sem=send_sems.at[lv, j],
                recv_sem=recv_sems.at[lv, j],
                device_id=pid,
                device_id_type=pl.DeviceIdType.MESH,
            )
            rdma.start()
            return rdma, nxt

        cpa.wait()
        cpb.wait()
        a16[...] = a_buf[...].astype(jnp.bfloat16)
        b16[...] = b_buf[...].astype(jnp.bfloat16)

        lo = [jnp.int32(0)] * NS
        R, keep = [None] * NS, [None] * NS
        for j in range(NS):
            acc[:, col[j]] = lax.dot_general(
                a16[...], b16[pl.ds(j * csz, csz), :],
                dimension_numbers=(((1,), (1,)), ((), ())),
                preferred_element_type=jnp.float32,
            ).astype(jnp.bfloat16)
            R[j], keep[j] = start_lvl(0, j, lo[j])

        for lv in range(4):
            rows = RS_ROWS[lv] * CHUNK
            off = RS_OFF[lv] * CHUNK
            for j in range(NS):
                R[j].wait_recv()
                if lv >= 2:
                    R[j].wait_send()
                else:
                    drain.append(R[j])
                lo[j] = keep[j]
                acc[pl.ds(lo[j] * CHUNK, rows), col[j]] += (
                    rbuf[pl.ds(off, rows), col[j]])
                R[j], keep[j] = start_lvl(lv + 1, j, lo[j])
                if lv == 3:
                    out_ref[pl.ds(lo[j] * CHUNK, rows), col[j]] = (
                        acc[pl.ds(lo[j] * CHUNK, rows), col[j]]
                        .astype(jnp.float32))

        for lv in (4, 5):
            seg = 4 if lv == 4 else 8
            for j in range(NS):
                R[j].wait_recv()
                drain.append(R[j])
                mybit = keep[j]
                recv_lo = lo[j] + (1 - 2 * mybit) * seg
                lo[j] = lo[j] - mybit * seg
                if lv == 4:
                    R[j], keep[j] = start_lvl(5, j, lo[j])
                out_ref[pl.ds(recv_lo * CHUNK, seg * CHUNK), col[j]] = (
                    acc[pl.ds(recv_lo * CHUNK, seg * CHUNK), col[j]]
                    .astype(jnp.float32))

        for rdma in drain:
            rdma.wait_send()

    return pl.pallas_call(
        body,
        out_shape=jax.ShapeDtypeStruct((m, d), jnp.float32),
        in_specs=[
            pl.BlockSpec(memory_space=pltpu.MemorySpace.HBM),
            pl.BlockSpec(memory_space=pltpu.MemorySpace.HBM),
        ],
        out_specs=pl.BlockSpec(memory_space=pltpu.VMEM),
        scratch_shapes=[
            pltpu.VMEM((m, slab), jnp.float32),
            pltpu.VMEM((m, slab), jnp.float32),
            pltpu.VMEM((m, slab), jnp.bfloat16),
            pltpu.VMEM((m, slab), jnp.bfloat16),
            pltpu.VMEM((m, d), jnp.bfloat16),
            pltpu.VMEM((20 * CHUNK, d), jnp.bfloat16),
            pltpu.SemaphoreType.DMA((2,)),
            pltpu.SemaphoreType.DMA((N_LVL, NS)),
            pltpu.SemaphoreType.DMA((N_LVL, NS)),
        ],
        compiler_params=pltpu.CompilerParams(
            collective_id=0,
            vmem_limit_bytes=100 * 1024 * 1024,
        ),
    )(dy, W)


# device time: 38492 ns/iter; 1.2141x vs baseline; 1.1546x over previous
import os

import jax
import jax.numpy as jnp
from jax import lax
from jax.experimental import pallas as pl
from jax.experimental.pallas import tpu as pltpu

N_SLABS = 8
NS = int(os.environ.get("KNS", "8"))
CHUNK = 64
N_LVL = 6
RS_OFF = (0, 8, 12, 16)
RS_ROWS = (8, 4, 4, 4)


def kernel(dy, W):
    m, k_shard = dy.shape
    d = W.shape[0]
    slab = k_shard // N_SLABS
    csz = d // NS

    def body(dy_hbm, w_hbm, out_ref, a_buf, b_buf, a16, b16, acc, rbuf,
             in_sems, send_sems, recv_sems):
        x = lax.axis_index("x")
        y = lax.axis_index("y")
        z = lax.axis_index("z")
        c = x * 4 + z

        cpa = pltpu.make_async_copy(
            dy_hbm.at[:, pl.ds(c * slab, slab)], a_buf, in_sems.at[0])
        cpb = pltpu.make_async_copy(
            w_hbm.at[:, pl.ds(c * slab, slab)], b_buf, in_sems.at[1])
        cpa.start()
        cpb.start()

        neighbors = [
            (1 - x, y, z),
            (x, 1 - y, z),
            (x, y, jnp.bitwise_xor(z, 1)),
            (x, y, jnp.bitwise_xor(z, 2)),
        ]
        bsem = pltpu.get_barrier_semaphore()
        for pid in neighbors:
            pl.semaphore_signal(bsem, inc=1, device_id=pid,
                                device_id_type=pl.DeviceIdType.MESH)
        pl.semaphore_wait(bsem, 4)

        info = {
            8: (neighbors[0], x),
            4: (neighbors[1], y),
            1: (neighbors[2], z & 1),
            2: (neighbors[3], z >> 1),
        }
        orders = ([
            (8, 4, 1, 2),
            (4, 8, 2, 1),
            (8, 4, 2, 1),
            (4, 8, 1, 2),
            (1, 2, 8, 4),
            (2, 1, 4, 8),
            (8, 4, 1, 2),
            (4, 8, 2, 1),
        ] * ((NS + 7) // 8))[:NS]
        col = [pl.ds(j * csz, csz) for j in range(NS)]
        lvl_mask = [
            [o[0], o[1], o[2], o[3], o[1], o[0]] for o in orders
        ]
        drain = []

        def start_lvl(lv, j, lo_j):
            pid, mybit = info[lvl_mask[j][lv]]
            if lv < 2:
                half = 8 >> lv
                send_lo = lo_j + (1 - mybit) * half
                keep_lo = lo_j + mybit * half
                src = acc.at[pl.ds(send_lo * CHUNK, half * CHUNK), col[j]]
                dst = rbuf.at[pl.ds(RS_OFF[lv] * CHUNK, half * CHUNK), col[j]]
                nxt = keep_lo
            elif lv < 4:
                src = acc.at[pl.ds(lo_j * CHUNK, 4 * CHUNK), col[j]]
                dst = rbuf.at[pl.ds(RS_OFF[lv] * CHUNK, 4 * CHUNK), col[j]]
                nxt = lo_j
            else:
                rows = 4 * CHUNK if lv == 4 else 8 * CHUNK
                src = acc.at[pl.ds(lo_j * CHUNK, rows), col[j]]
                dst = acc.at[pl.ds(lo_j * CHUNK, rows), col[j]]
                nxt = mybit
            rdma = pltpu.make_async_remote_copy(
                src_ref=src, dst_ref=dst,
                send_sem=send_sems.at[lv, j],
                recv_sem=recv_sems.at[lv, j],
                device_id=pid,
                device_id_type=pl.DeviceIdType.MESH,
            )
            rdma.start()
            return rdma, nxt

        cpa.wait()
        cpb.wait()
        a16[...] = a_buf[...].astype(jnp.bfloat16)
        b16[...] = b_buf[...].astype(jnp.bfloat16)

        lo = [jnp.int32(0)] * NS
        R, keep = [None] * NS, [None] * NS
        for j in range(NS):
            acc[:, col[j]] = lax.dot_general(
                a16[...], b16[pl.ds(j * csz, csz), :],
                dimension_numbers=(((1,), (1,)), ((), ())),
                preferred_element_type=jnp.float32,
            ).astype(jnp.bfloat16)
            R[j], keep[j] = start_lvl(0, j, lo[j])

        for lv in range(4):
            rows = RS_ROWS[lv] * CHUNK
            off = RS_OFF[lv] * CHUNK
            for j in range(NS):
                R[j].wait_recv()
                if lv >= 2:
                    R[j].wait_send()
                else:
                    drain.append(R[j])
                lo[j] = keep[j]
                acc[pl.ds(lo[j] * CHUNK, rows), col[j]] += (
                    rbuf[pl.ds(off, rows), col[j]])
                R[j], keep[j] = start_lvl(lv + 1, j, lo[j])

        for lv in (4, 5):
            seg = 4 if lv == 4 else 8
            for j in range(NS):
                R[j].wait_recv()
                drain.append(R[j])
                mybit = keep[j]
                lo[j] = lo[j] - mybit * seg
                if lv == 4:
                    R[j], keep[j] = start_lvl(5, j, lo[j])

        for rdma in drain:
            rdma.wait_send()
        out_ref[...] = acc[...].astype(jnp.float32)

    return pl.pallas_call(
        body,
        out_shape=jax.ShapeDtypeStruct((m, d), jnp.float32),
        in_specs=[
            pl.BlockSpec(memory_space=pltpu.MemorySpace.HBM),
            pl.BlockSpec(memory_space=pltpu.MemorySpace.HBM),
        ],
        out_specs=pl.BlockSpec(memory_space=pltpu.VMEM),
        scratch_shapes=[
            pltpu.VMEM((m, slab), jnp.float32),
            pltpu.VMEM((m, slab), jnp.float32),
            pltpu.VMEM((m, slab), jnp.bfloat16),
            pltpu.VMEM((m, slab), jnp.bfloat16),
            pltpu.VMEM((m, d), jnp.bfloat16),
            pltpu.VMEM((20 * CHUNK, d), jnp.bfloat16),
            pltpu.SemaphoreType.DMA((2,)),
            pltpu.SemaphoreType.DMA((N_LVL, NS)),
            pltpu.SemaphoreType.DMA((N_LVL, NS)),
        ],
        compiler_params=pltpu.CompilerParams(
            collective_id=0,
            vmem_limit_bytes=100 * 1024 * 1024,
        ),
    )(dy, W)
